# baseline (speedup 1.0000x reference)
import jax
import jax.numpy as jnp
from jax import lax
from jax.experimental import pallas as pl
from jax.experimental.pallas import tpu as pltpu
from jax.experimental.pallas import tpu_sc as plsc

_NC = 2
_NS = 16
_NW = _NC * _NS
_L = 16
_NB1 = 512
_HS = 513
_UW = 2049


def _body(scores_hbm, karr_hbm, out_hbm, buf0, buf1, ubuf, hist, kbuf,
          isem0, isem1, osem0, osem1):
    rows, cols = scores_hbm.shape
    rpw = rows // _NW
    nvr = cols // _L
    wid = lax.axis_index("s") * _NC + lax.axis_index("c")

    pltpu.sync_copy(karr_hbm, kbuf)
    kk = kbuf[...][0]

    lane = lax.iota(jnp.int32, _L)
    zeros16 = jnp.zeros((_L,), jnp.int32)
    ones16 = jnp.ones((_L,), jnp.int32)
    lane_base = lane * _HS
    lane_cw = lane * _UW

    def _zh(i, c):
        hist[pl.ds(i * _L, _L)] = zeros16
        return c

    lax.fori_loop(0, _HS * _L // _L + 1, _zh, 0, unroll=8)

    def _scan_hist(kk_rem, nb):
        def g_body(g, carry):
            running, bin_found, before = carry
            parts = [hist[pl.ds(l * _HS + g, _L)] for l in range(_L)]
            while len(parts) > 1:
                parts = [a + b for a, b in zip(parts[::2], parts[1::2])]
            tot = parts[0]
            for l in range(_L):
                hist[pl.ds(l * _HS + g, _L)] = zeros16
            cs = plsc.cumsum(tot)
            cum = running + cs
            m = cum >= kk_rem
            cnt_ge = plsc.all_reduce_population_count(m)[0]
            found_now = jnp.logical_and(cnt_ge > 0, bin_found < 0)
            firstlane = _L - cnt_ge
            grp_tot = cs[_L - 1]
            before_cand = running + (grp_tot - jnp.sum(jnp.where(m, tot, 0)))
            bin_found = jnp.where(found_now, g + firstlane, bin_found)
            before = jnp.where(found_now, before_cand, before)
            return running + grp_tot, bin_found, before

        init = (jnp.int32(0), jnp.int32(-1), jnp.int32(0))
        _, b, before = plsc.parallel_loop(0, nb, _L, carry=init)(g_body)
        return b, before

    def _hist_pass(rowbuf):
        @plsc.parallel_loop(0, cols, _L, unroll=16)
        def _h1(i):
            v = rowbuf[pl.ds(i, _L)]
            u = plsc.bitcast(jnp.abs(v), jnp.int32)
            plsc.addupdate_scatter(
                hist, [lane_base + lax.shift_right_logical(u, 22)], ones16)

    def _select_and_mask(rowbuf):
        bin1, before1 = _scan_hist(kk, _NB1)
        kk2 = kk - before1

        def cpass(i, cnt_vec):
            v = rowbuf[pl.ds(i, _L)]
            u = plsc.bitcast(jnp.abs(v), jnp.int32)
            m = lax.shift_right_logical(u, 22) == bin1
            plsc.store_scatter(ubuf, [lane_cw + cnt_vec], u, mask=m)
            return cnt_vec + m.astype(jnp.int32)

        cnt_vec = plsc.parallel_loop(0, cols, _L, unroll=16, carry=zeros16)(cpass)
        maxcnt = jnp.max(cnt_vec)

        def cand_hist(shift, bmask, pshift, prefix):
            @plsc.parallel_loop(0, maxcnt, unroll=8)
            def _hb(j):
                idx = lane_cw + j
                uv = plsc.load_gather(ubuf, [idx])
                m = jnp.logical_and(
                    j < cnt_vec,
                    lax.shift_right_logical(uv, pshift) == prefix)
                b = jnp.bitwise_and(lax.shift_right_logical(uv, shift), bmask)
                plsc.addupdate_scatter(hist, [lane_base + b], ones16, mask=m)

        cand_hist(14, 0xFF, 22, bin1)
        bin2, before2 = _scan_hist(kk2, 256)
        kk3 = kk2 - before2
        pre2 = (bin1 << 8) | bin2

        cand_hist(6, 0xFF, 14, pre2)
        bin3, before3 = _scan_hist(kk3, 256)
        kk4 = kk3 - before3
        pre3 = (pre2 << 8) | bin3

        cand_hist(0, 0x3F, 6, pre3)
        bin4, _ = _scan_hist(kk4, 64)

        t = (pre3 << 6) | bin4
        t = jnp.where(kk > 0, t, -1)

        @plsc.parallel_loop(0, cols, _L, unroll=16)
        def _mp(i):
            v = rowbuf[pl.ds(i, _L)]
            u = plsc.bitcast(jnp.abs(v), jnp.int32)
            rowbuf[pl.ds(i, _L)] = jnp.where(u <= t, 0.0, v)

    bufs = (buf0, buf1)
    isems = (isem0, isem1)
    osems = (osem0, osem1)
    row0 = wid * rpw

    copies_in = [None] * rpw
    copies_out = [None] * rpw
    copies_in[0] = pltpu.async_copy(scores_hbm.at[row0], buf0, isem0)
    for r in range(rpw):
        p = r % 2
        copies_in[r].wait()
        _hist_pass(bufs[p])
        if r + 1 < rpw:
            if r >= 1:
                copies_out[r - 1].wait()
            copies_in[r + 1] = pltpu.async_copy(
                scores_hbm.at[row0 + r + 1], bufs[1 - p], isems[1 - p])
        _select_and_mask(bufs[p])
        copies_out[r] = pltpu.async_copy(
            bufs[p], out_hbm.at[row0 + r], osems[p])
    copies_out[rpw - 2].wait()
    copies_out[rpw - 1].wait()


def kernel(scores, k):
    rows, cols = scores.shape
    kk = jnp.clip(jnp.asarray(k, jnp.int32), 0, cols // 10)
    karr = jnp.zeros((_L,), jnp.int32).at[0].set(kk)
    mesh = plsc.VectorSubcoreMesh(
        core_axis_name="c", subcore_axis_name="s",
        num_cores=_NC, num_subcores=_NS)
    f = pl.kernel(
        _body,
        out_type=jax.ShapeDtypeStruct((rows, cols), jnp.float32),
        mesh=mesh,
        scratch_types=[
            pltpu.VMEM((cols,), jnp.float32),
            pltpu.VMEM((cols,), jnp.float32),
            pltpu.VMEM((_UW * _L,), jnp.int32),
            pltpu.VMEM((_HS * _L + _L,), jnp.int32),
            pltpu.VMEM((_L,), jnp.int32),
            pltpu.SemaphoreType.DMA,
            pltpu.SemaphoreType.DMA,
            pltpu.SemaphoreType.DMA,
            pltpu.SemaphoreType.DMA,
        ],
        compiler_params=pltpu.CompilerParams(needs_layout_passes=False),
    )
    return f(scores, karr)


# speedup vs baseline: 1.0051x; 1.0051x over previous
import jax
import jax.numpy as jnp
from jax import lax
from jax.experimental import pallas as pl
from jax.experimental.pallas import tpu as pltpu
from jax.experimental.pallas import tpu_sc as plsc

_NC = 2
_NS = 16
_NW = _NC * _NS
_L = 16
_NB1 = 512
_HS = 513
_UW = 2049


def _body(scores_hbm, karr_hbm, out_hbm, buf0, buf1, ubuf, hist, kbuf,
          isem0, isem1, osem0, osem1):
    rows, cols = scores_hbm.shape
    rpw = rows // _NW
    nvr = cols // _L
    wid = lax.axis_index("s") * _NC + lax.axis_index("c")

    pltpu.sync_copy(karr_hbm, kbuf)
    kk = kbuf[...][0]

    lane = lax.iota(jnp.int32, _L)
    zeros16 = jnp.zeros((_L,), jnp.int32)
    ones16 = jnp.ones((_L,), jnp.int32)
    lane_base = lane * _HS
    lane_cw = lane * _UW

    def _zh(i, c):
        hist[pl.ds(i * _L, _L)] = zeros16
        return c

    lax.fori_loop(0, _HS * _L // _L + 1, _zh, 0, unroll=8)

    def _scan_hist(kk_rem, nb):
        def g_body(g, carry):
            running, bin_found, before = carry
            parts = [hist[pl.ds(l * _HS + g, _L)] for l in range(_L)]
            while len(parts) > 1:
                parts = [a + b for a, b in zip(parts[::2], parts[1::2])]
            tot = parts[0]
            for l in range(_L):
                hist[pl.ds(l * _HS + g, _L)] = zeros16
            cs = plsc.cumsum(tot)
            cum = running + cs
            m = cum >= kk_rem
            cnt_ge = plsc.all_reduce_population_count(m)[0]
            found_now = jnp.logical_and(cnt_ge > 0, bin_found < 0)
            firstlane = _L - cnt_ge
            grp_tot = cs[_L - 1]
            before_cand = running + (grp_tot - jnp.sum(jnp.where(m, tot, 0)))
            bin_found = jnp.where(found_now, g + firstlane, bin_found)
            before = jnp.where(found_now, before_cand, before)
            return running + grp_tot, bin_found, before

        init = (jnp.int32(0), jnp.int32(-1), jnp.int32(0))
        _, b, before = plsc.parallel_loop(0, nb, _L, carry=init)(g_body)
        return b, before

    def _hist_pass(rowbuf):
        @plsc.parallel_loop(0, cols, _L, unroll=16)
        def _h1(i):
            v = rowbuf[pl.ds(i, _L)]
            u = plsc.bitcast(jnp.abs(v), jnp.int32)
            plsc.addupdate_scatter(
                hist, [lane_base + lax.shift_right_logical(u, 22)], ones16)

    def _select_and_mask(rowbuf):
        bin1, before1 = _scan_hist(kk, _NB1)
        kk2 = kk - before1

        def cpass(i, cnt_vec):
            v = rowbuf[pl.ds(i, _L)]
            u = plsc.bitcast(jnp.abs(v), jnp.int32)
            m = lax.shift_right_logical(u, 22) == bin1
            plsc.store_scatter(ubuf, [lane_cw + cnt_vec], u, mask=m)
            return cnt_vec + m.astype(jnp.int32)

        cnt_vec = plsc.parallel_loop(0, cols, _L, unroll=8, carry=zeros16)(cpass)
        maxcnt = jnp.max(cnt_vec)

        def cand_hist(shift, bmask, pshift, prefix):
            @plsc.parallel_loop(0, maxcnt, unroll=8)
            def _hb(j):
                idx = lane_cw + j
                uv = plsc.load_gather(ubuf, [idx])
                m = jnp.logical_and(
                    j < cnt_vec,
                    lax.shift_right_logical(uv, pshift) == prefix)
                b = jnp.bitwise_and(lax.shift_right_logical(uv, shift), bmask)
                plsc.addupdate_scatter(hist, [lane_base + b], ones16, mask=m)

        cand_hist(14, 0xFF, 22, bin1)
        bin2, before2 = _scan_hist(kk2, 256)
        kk3 = kk2 - before2
        pre2 = (bin1 << 8) | bin2

        cand_hist(6, 0xFF, 14, pre2)
        bin3, before3 = _scan_hist(kk3, 256)
        kk4 = kk3 - before3
        pre3 = (pre2 << 8) | bin3

        cand_hist(0, 0x3F, 6, pre3)
        bin4, _ = _scan_hist(kk4, 64)

        t = (pre3 << 6) | bin4
        t = jnp.where(kk > 0, t, -1)

        @plsc.parallel_loop(0, cols, _L, unroll=16)
        def _mp(i):
            v = rowbuf[pl.ds(i, _L)]
            u = plsc.bitcast(jnp.abs(v), jnp.int32)
            rowbuf[pl.ds(i, _L)] = jnp.where(u <= t, 0.0, v)

    bufs = (buf0, buf1)
    isems = (isem0, isem1)
    osems = (osem0, osem1)
    row0 = wid * rpw

    copies_in = [None] * rpw
    copies_out = [None] * rpw
    copies_in[0] = pltpu.async_copy(scores_hbm.at[row0], buf0, isem0)
    for r in range(rpw):
        p = r % 2
        copies_in[r].wait()
        _hist_pass(bufs[p])
        if r + 1 < rpw:
            if r >= 1:
                copies_out[r - 1].wait()
            copies_in[r + 1] = pltpu.async_copy(
                scores_hbm.at[row0 + r + 1], bufs[1 - p], isems[1 - p])
        _select_and_mask(bufs[p])
        copies_out[r] = pltpu.async_copy(
            bufs[p], out_hbm.at[row0 + r], osems[p])
    copies_out[rpw - 2].wait()
    copies_out[rpw - 1].wait()


def kernel(scores, k):
    rows, cols = scores.shape
    kk = jnp.clip(jnp.asarray(k, jnp.int32), 0, cols // 10)
    karr = jnp.zeros((_L,), jnp.int32).at[0].set(kk)
    mesh = plsc.VectorSubcoreMesh(
        core_axis_name="c", subcore_axis_name="s",
        num_cores=_NC, num_subcores=_NS)
    f = pl.kernel(
        _body,
        out_type=jax.ShapeDtypeStruct((rows, cols), jnp.float32),
        mesh=mesh,
        scratch_types=[
            pltpu.VMEM((cols,), jnp.float32),
            pltpu.VMEM((cols,), jnp.float32),
            pltpu.VMEM((_UW * _L,), jnp.int32),
            pltpu.VMEM((_HS * _L + _L,), jnp.int32),
            pltpu.VMEM((_L,), jnp.int32),
            pltpu.SemaphoreType.DMA,
            pltpu.SemaphoreType.DMA,
            pltpu.SemaphoreType.DMA,
            pltpu.SemaphoreType.DMA,
        ],
        compiler_params=pltpu.CompilerParams(needs_layout_passes=False),
    )
    return f(scores, karr)
